# baseline (speedup 1.0000x reference)
import dataclasses
import functools

import jax
import jax.numpy as jnp
import numpy as np
from jax import lax
from jax.experimental import pallas as pl
from jax.experimental.pallas import tpu as pltpu
from jax.experimental.pallas import tpu_sc as plsc

N = 10000
DEG = 32
IN_DIM = 128
HID = 64
H1 = 8
C = 40
NEG = 0.2

D1 = H1 * HID
D2 = 48
SL1 = 6
R1 = SL1 * 128
R2 = 128

NWORKERS = 32
NP_ = 10240
SEG_PER_W = NP_ // NWORKERS
INNER = 16
OUTER = SEG_PER_W // INNER
LANES = 16


def _mm_body(x_ref, wf_ref, wle_ref, f_ref, le_ref):
    x = x_ref[...]
    f_ref[...] = jnp.dot(
        x, wf_ref[...], preferred_element_type=jnp.float32
    ).astype(f_ref.dtype)
    le_ref[...] = jnp.dot(x, wle_ref[...], preferred_element_type=jnp.float32)


def _mm(x, wf, wle, bm=512, f_dtype=jnp.float32):
    n, k = x.shape
    df, dle = wf.shape[1], wle.shape[1]
    return pl.pallas_call(
        _mm_body,
        grid=(n // bm,),
        in_specs=[
            pl.BlockSpec((bm, k), lambda i: (i, 0)),
            pl.BlockSpec((k, df), lambda i: (0, 0)),
            pl.BlockSpec((k, dle), lambda i: (0, 0)),
        ],
        out_specs=[
            pl.BlockSpec((bm, df), lambda i: (i, 0)),
            pl.BlockSpec((bm, dle), lambda i: (i, 0)),
        ],
        out_shape=[
            jax.ShapeDtypeStruct((n, df), f_dtype),
            jax.ShapeDtypeStruct((n, dle), jnp.float32),
        ],
    )(x, wf, wle)


CSMAX = 448


def _core_split(cs0):
    cc = lax.axis_index("c")
    ss = lax.axis_index("s")
    cs = jnp.where(cc == 0, cs0, 2 * SEG_PER_W - cs0)
    base = cc * 16 * cs0 + ss * cs
    return base, cs


def _make_sc_edge(sl, el_off, d_out, NBUF, cs0):
    nv = d_out // LANES
    fgat_shape = (NBUF, DEG, sl, 128) if sl > 1 else (NBUF, DEG, 128)
    mesh = plsc.VectorSubcoreMesh(core_axis_name="c", subcore_axis_name="s")

    @functools.partial(
        pl.kernel,
        out_type=jax.ShapeDtypeStruct((NP_, d_out), jnp.float32),
        mesh=mesh,
        scratch_types=[
            pltpu.VMEM((CSMAX, DEG), jnp.int32),
            pltpu.VMEM((INNER, 32), jnp.float32),
            pltpu.VMEM(fgat_shape, jnp.float32),
            pltpu.VMEM((DEG, LANES), jnp.float32),
            pltpu.VMEM((INNER, d_out), jnp.float32),
        ] + [pltpu.SemaphoreType.DMA] * NBUF,
    )
    def sc_edge(f_hbm, le_hbm, idx_hbm, out_hbm,
                idx_v, ler_v, fgat, exbuf, obuf, *sems):
        base, cs = _core_split(cs0)
        pltpu.sync_copy(idx_hbm.at[pl.ds(base, CSMAX)], idx_v)

        def gather(s, b):
            return pltpu.make_async_copy(
                f_hbm.at[idx_v.at[s]], fgat.at[b], sems[b])

        for b in range(NBUF - 1):
            gather(b, b).start()

        @pl.loop(0, cs // INNER)
        def _outer(k):
            pltpu.sync_copy(le_hbm.at[pl.ds(base + k * INNER, INNER)], ler_v)

            @pl.loop(0, INNER // NBUF)
            def _inner(jj):
                for b in range(NBUF):
                    j = jj * NBUF + b
                    s = k * INNER + j
                    gather(s, b).wait()
                    s_next = s + NBUF - 1

                    @pl.when(s_next < cs)
                    def _pref():
                        gather(s_next, (b + NBUF - 1) % NBUF).start()

                    er = ler_v[j, pl.ds(LANES, LANES)]

                    def frow(bb, e, flat):
                        if sl > 1:
                            return fgat[bb, e, flat // 128,
                                        pl.ds(flat % 128, LANES)]
                        return fgat[bb, e, pl.ds(flat, LANES)]

                    def p1(e, m):
                        t = frow(b, e, el_off) + er
                        return jnp.maximum(m, jnp.maximum(t, NEG * t))

                    m = lax.fori_loop(0, DEG, p1,
                                      jnp.full((LANES,), -3e38, jnp.float32))

                    def p2(e, ssum):
                        t = frow(b, e, el_off) + er
                        ex = jnp.exp(jnp.maximum(t, NEG * t) - m)
                        exbuf[e, pl.ds(0, LANES)] = ex
                        return ssum + ex

                    ssum = lax.fori_loop(0, DEG, p2,
                                         jnp.zeros((LANES,), jnp.float32))
                    inv = 1.0 / (ssum + 1e-9)

                    def p3(e, accs):
                        alpha = exbuf[e, pl.ds(0, LANES)] * inv
                        return tuple(
                            accs[c] + alpha * frow(b, e, c * LANES)
                            for c in range(nv))

                    accs = lax.fori_loop(
                        0, DEG, p3,
                        tuple(jnp.zeros((LANES,), jnp.float32)
                              for _ in range(nv)))
                    for c in range(nv):
                        obuf[j, pl.ds(c * LANES, LANES)] = accs[c]

            pltpu.sync_copy(obuf,
                            out_hbm.at[pl.ds(base + k * INNER, INNER)])

    return sc_edge


def _bf2f32(t16):
    va = jax.lax.bitcast_convert_type(t16 << 16, jnp.float32)
    vb = jax.lax.bitcast_convert_type(t16 & jnp.int32(-65536), jnp.float32)
    return va, vb


def _make_sc_edge_l1_bf16(NBUF, cs0, INNER=8):
    mesh = plsc.VectorSubcoreMesh(core_axis_name="c", subcore_axis_name="s")

    @functools.partial(
        pl.kernel,
        out_type=jax.ShapeDtypeStruct((NP_, D1), jnp.float32),
        mesh=mesh,
        scratch_types=[
            pltpu.VMEM((CSMAX, DEG), jnp.int32),
            pltpu.VMEM((INNER, 3, 128), jnp.int32),
            pltpu.VMEM((NBUF, DEG, 3, 128), jnp.int32),
            pltpu.VMEM((DEG, LANES), jnp.float32),
            pltpu.VMEM((INNER, D1), jnp.float32),
        ] + [pltpu.SemaphoreType.DMA] * NBUF,
    )
    def sc_edge(f_hbm, idx_hbm, out_hbm,
                idx_v, ler_v, fgat, exbuf, obuf, *sems):
        base, cs = _core_split(cs0)
        pltpu.sync_copy(idx_hbm.at[pl.ds(base, CSMAX)], idx_v)

        def gat_f(s, b):
            return pltpu.make_async_copy(
                f_hbm.at[idx_v.at[s]], fgat.at[b], sems[b])

        for b in range(NBUF - 1):
            gat_f(b, b).start()

        @pl.loop(0, cs // INNER)
        def _outer(k):
            pltpu.sync_copy(f_hbm.at[pl.ds(base + k * INNER, INNER)], ler_v)

            @pl.loop(0, INNER // NBUF)
            def _inner(jj):
                for b in range(NBUF):
                    j = jj * NBUF + b
                    s = k * INNER + j
                    gat_f(s, b).wait()
                    s_next = s + NBUF - 1

                    @pl.when(s_next < cs)
                    def _pref():
                        gat_f(s_next, (b + NBUF - 1) % NBUF).start()

                    er_hi, er_lo = _bf2f32(ler_v[j, 2, pl.ds(LANES, LANES)])
                    er = er_hi + er_lo

                    def p1(e, m):
                        el_hi, el_lo = _bf2f32(fgat[b, e, 2, pl.ds(0, LANES)])
                        t = el_hi + el_lo + er
                        return jnp.maximum(m, jnp.maximum(t, NEG * t))

                    m = lax.fori_loop(0, DEG, p1,
                                      jnp.full((LANES,), -3e38, jnp.float32))

                    def p2(e, ssum):
                        el_hi, el_lo = _bf2f32(fgat[b, e, 2, pl.ds(0, LANES)])
                        t = el_hi + el_lo + er
                        ex = jnp.exp(jnp.maximum(t, NEG * t) - m)
                        exbuf[e, pl.ds(0, LANES)] = ex
                        return ssum + ex

                    ssum = lax.fori_loop(0, DEG, p2,
                                         jnp.zeros((LANES,), jnp.float32))
                    inv = 1.0 / (ssum + 1e-9)

                    def p3(e, accs):
                        alpha = exbuf[e, pl.ds(0, LANES)] * inv
                        out = list(accs)
                        for g in range(16):
                            t16 = fgat[b, e, g // 8,
                                       pl.ds((g % 8) * LANES, LANES)]
                            va, vb = _bf2f32(t16)
                            out[2 * g] = out[2 * g] + alpha * va
                            out[2 * g + 1] = out[2 * g + 1] + alpha * vb
                        return tuple(out)

                    accs = lax.fori_loop(
                        0, DEG, p3,
                        tuple(jnp.zeros((LANES,), jnp.float32)
                              for _ in range(32)))
                    for c in range(32):
                        obuf[j, pl.ds(c * LANES, LANES)] = accs[c]

            pltpu.sync_copy(obuf,
                            out_hbm.at[pl.ds(base + k * INNER, INNER)])

    return sc_edge


CS0_L1 = 224
CS0_L2 = 224

_sc_edge_l1 = _make_sc_edge_l1_bf16(2, CS0_L1)
_sc_edge_l2 = _make_sc_edge(1, D2, D2, 8, CS0_L2)


def kernel(inputs, row_ptr, col_idx, col_ptr, row_idx, permute,
           W1, al1, ar1, W2, al2, ar2):
    xp = jnp.pad(inputs, ((0, NP_ - N), (0, 0)))
    idx = jnp.pad(
        col_idx, (0, (NP_ + CSMAX) * DEG - N * DEG)).reshape(NP_ + CSMAX, DEG)

    t = np.arange(D1)
    g_, r_ = t // 32, t % 32
    p_, l_ = r_ % 2, r_ // 2
    perm_table = (l_ % 8) * HID + (4 * g_ + 2 * p_ + (l_ >= 8))
    jo = np.arange(D1)
    v_, lo_ = jo // LANES, jo % LANES
    perm_out = (lo_ % 8) * HID + (4 * (v_ // 2) + 2 * (v_ % 2) + (lo_ >= 8))

    w1r = W1.reshape(IN_DIM, H1, HID)
    A1 = jnp.einsum("khd,hd->kh", w1r, al1)
    B1 = jnp.einsum("khd,hd->kh", w1r, ar1)
    WF1 = W1[:, perm_table]
    WLE1 = jnp.concatenate([A1, A1, B1, B1], axis=1)

    F1, LE1 = _mm(xp, WF1, WLE1, f_dtype=jnp.bfloat16)
    F1i = jax.lax.bitcast_convert_type(
        F1.reshape(NP_, D1 // 2, 2), jnp.int32)

    def _hilo(x):
        hi = x.astype(jnp.bfloat16)
        lo = (x - hi.astype(jnp.float32)).astype(jnp.bfloat16)
        return jax.lax.bitcast_convert_type(
            jnp.stack([hi, lo], axis=-1), jnp.int32)

    tab1 = jnp.concatenate(
        [F1i, _hilo(LE1[:, :16]), _hilo(LE1[:, 16:]),
         jnp.zeros((NP_, 96), jnp.int32)], axis=1).reshape(NP_, 3, 128)
    H1o = _sc_edge_l1(tab1, idx)

    W2p = W2[perm_out, :]
    A2 = W2p @ al2[0]
    B2 = W2p @ ar2[0]
    W2p = jnp.pad(W2p, ((0, 0), (0, D2 - C)))
    A2t = jnp.tile(A2[:, None], (1, LANES))
    B2t = jnp.tile(B2[:, None], (1, LANES))
    WF2 = jnp.concatenate(
        [W2p, A2t, B2t,
         jnp.zeros((D1, R2 - D2 - 2 * LANES), jnp.float32)], axis=1)
    WLE2 = jnp.concatenate([A2t, B2t], axis=1)

    F2, LE2 = _mm(H1o, WF2, WLE2)
    H2 = _sc_edge_l2(F2, LE2, idx)
    return H2[:N, :C]


# speedup vs baseline: 1.2220x; 1.2220x over previous
import dataclasses
import functools

import jax
import jax.numpy as jnp
import numpy as np
from jax import lax
from jax.experimental import pallas as pl
from jax.experimental.pallas import tpu as pltpu
from jax.experimental.pallas import tpu_sc as plsc

N = 10000
DEG = 32
IN_DIM = 128
HID = 64
H1 = 8
C = 40
NEG = 0.2

D1 = H1 * HID
D2 = 48
SL1 = 6
R1 = SL1 * 128
R2 = 128

NWORKERS = 32
NP_ = 10240
SEG_PER_W = NP_ // NWORKERS
INNER = 16
OUTER = SEG_PER_W // INNER
LANES = 16


def _mm_body(x_ref, wf_ref, wle_ref, f_ref, le_ref):
    x = x_ref[...]
    f_ref[...] = jnp.dot(
        x, wf_ref[...], preferred_element_type=jnp.float32
    ).astype(f_ref.dtype)
    le_ref[...] = jnp.dot(x, wle_ref[...], preferred_element_type=jnp.float32)


def _mm(x, wf, wle, bm=512, f_dtype=jnp.float32):
    n, k = x.shape
    df, dle = wf.shape[1], wle.shape[1]
    return pl.pallas_call(
        _mm_body,
        grid=(n // bm,),
        in_specs=[
            pl.BlockSpec((bm, k), lambda i: (i, 0)),
            pl.BlockSpec((k, df), lambda i: (0, 0)),
            pl.BlockSpec((k, dle), lambda i: (0, 0)),
        ],
        out_specs=[
            pl.BlockSpec((bm, df), lambda i: (i, 0)),
            pl.BlockSpec((bm, dle), lambda i: (i, 0)),
        ],
        out_shape=[
            jax.ShapeDtypeStruct((n, df), f_dtype),
            jax.ShapeDtypeStruct((n, dle), jnp.float32),
        ],
    )(x, wf, wle)


CSMAX = 448


def _core_split(cs0):
    cc = lax.axis_index("c")
    ss = lax.axis_index("s")
    cs = jnp.where(cc == 0, cs0, 2 * SEG_PER_W - cs0)
    base = cc * 16 * cs0 + ss * cs
    return base, cs


def _make_sc_edge(sl, el_off, d_out, NBUF, cs0):
    nv = d_out // LANES
    fgat_shape = (NBUF, DEG, sl, 128) if sl > 1 else (NBUF, DEG, 128)
    mesh = plsc.VectorSubcoreMesh(core_axis_name="c", subcore_axis_name="s")

    @functools.partial(
        pl.kernel,
        out_type=jax.ShapeDtypeStruct((NP_, d_out), jnp.float32),
        mesh=mesh,
        scratch_types=[
            pltpu.VMEM((CSMAX, DEG), jnp.int32),
            pltpu.VMEM((INNER, 32), jnp.float32),
            pltpu.VMEM(fgat_shape, jnp.float32),
            pltpu.VMEM((DEG, LANES), jnp.float32),
            pltpu.VMEM((INNER, d_out), jnp.float32),
        ] + [pltpu.SemaphoreType.DMA] * NBUF,
    )
    def sc_edge(f_hbm, le_hbm, idx_hbm, out_hbm,
                idx_v, ler_v, fgat, exbuf, obuf, *sems):
        base, cs = _core_split(cs0)
        pltpu.sync_copy(idx_hbm.at[pl.ds(base, CSMAX)], idx_v)

        def gather(s, b):
            return pltpu.make_async_copy(
                f_hbm.at[idx_v.at[s]], fgat.at[b], sems[b])

        for b in range(NBUF - 1):
            gather(b, b).start()

        @pl.loop(0, cs // INNER)
        def _outer(k):
            pltpu.sync_copy(le_hbm.at[pl.ds(base + k * INNER, INNER)], ler_v)

            @pl.loop(0, INNER // NBUF)
            def _inner(jj):
                for b in range(NBUF):
                    j = jj * NBUF + b
                    s = k * INNER + j
                    gather(s, b).wait()
                    s_next = s + NBUF - 1

                    @pl.when(s_next < cs)
                    def _pref():
                        gather(s_next, (b + NBUF - 1) % NBUF).start()

                    er = ler_v[j, pl.ds(LANES, LANES)]

                    def frow(bb, e, flat):
                        if sl > 1:
                            return fgat[bb, e, flat // 128,
                                        pl.ds(flat % 128, LANES)]
                        return fgat[bb, e, pl.ds(flat, LANES)]

                    def p1(e, m):
                        t = frow(b, e, el_off) + er
                        return jnp.maximum(m, jnp.maximum(t, NEG * t))

                    m = lax.fori_loop(0, DEG, p1,
                                      jnp.full((LANES,), -3e38, jnp.float32))

                    def p2(e, ssum):
                        t = frow(b, e, el_off) + er
                        ex = jnp.exp(jnp.maximum(t, NEG * t) - m)
                        exbuf[e, pl.ds(0, LANES)] = ex
                        return ssum + ex

                    ssum = lax.fori_loop(0, DEG, p2,
                                         jnp.zeros((LANES,), jnp.float32))
                    inv = 1.0 / (ssum + 1e-9)

                    def p3(e, accs):
                        alpha = exbuf[e, pl.ds(0, LANES)] * inv
                        return tuple(
                            accs[c] + alpha * frow(b, e, c * LANES)
                            for c in range(nv))

                    accs = lax.fori_loop(
                        0, DEG, p3,
                        tuple(jnp.zeros((LANES,), jnp.float32)
                              for _ in range(nv)))
                    for c in range(nv):
                        obuf[j, pl.ds(c * LANES, LANES)] = accs[c]

            pltpu.sync_copy(obuf,
                            out_hbm.at[pl.ds(base + k * INNER, INNER)])

    return sc_edge


def _bf2f32(t16):
    va = jax.lax.bitcast_convert_type(t16 << 16, jnp.float32)
    vb = jax.lax.bitcast_convert_type(t16 & jnp.int32(-65536), jnp.float32)
    return va, vb


def _make_sc_edge_l1_bf16(NBUF, cs0, INNER=8):
    mesh = plsc.VectorSubcoreMesh(core_axis_name="c", subcore_axis_name="s")

    @functools.partial(
        pl.kernel,
        out_type=jax.ShapeDtypeStruct((NP_, D1), jnp.float32),
        mesh=mesh,
        scratch_types=[
            pltpu.VMEM((CSMAX, DEG), jnp.int32),
            pltpu.VMEM((INNER, 3, 128), jnp.int32),
            pltpu.VMEM((NBUF, DEG, 3, 128), jnp.int32),
            pltpu.VMEM((DEG, LANES), jnp.float32),
            pltpu.VMEM((INNER, D1), jnp.float32),
        ] + [pltpu.SemaphoreType.DMA] * NBUF,
    )
    def sc_edge(f_hbm, idx_hbm, out_hbm,
                idx_v, ler_v, fgat, exbuf, obuf, *sems):
        base, cs = _core_split(cs0)
        pltpu.sync_copy(idx_hbm.at[pl.ds(base, CSMAX)], idx_v)

        def gat_f(s, b):
            return pltpu.make_async_copy(
                f_hbm.at[idx_v.at[s]], fgat.at[b], sems[b])

        for b in range(NBUF - 1):
            gat_f(b, b).start()

        @pl.loop(0, cs // INNER)
        def _outer(k):
            pltpu.sync_copy(f_hbm.at[pl.ds(base + k * INNER, INNER)], ler_v)

            @pl.loop(0, INNER // NBUF)
            def _inner(jj):
                for b in range(NBUF):
                    j = jj * NBUF + b
                    s = k * INNER + j
                    gat_f(s, b).wait()
                    s_next = s + NBUF - 1

                    @pl.when(s_next < cs)
                    def _pref():
                        gat_f(s_next, (b + NBUF - 1) % NBUF).start()

                    er_hi, er_lo = _bf2f32(ler_v[j, 2, pl.ds(LANES, LANES)])
                    er = er_hi + er_lo

                    def p1(e, m):
                        el_hi, el_lo = _bf2f32(fgat[b, e, 2, pl.ds(0, LANES)])
                        t = el_hi + el_lo + er
                        return jnp.maximum(m, jnp.maximum(t, NEG * t))

                    m = lax.fori_loop(0, DEG, p1,
                                      jnp.full((LANES,), -3e38, jnp.float32))

                    def p2(e, ssum):
                        el_hi, el_lo = _bf2f32(fgat[b, e, 2, pl.ds(0, LANES)])
                        t = el_hi + el_lo + er
                        ex = jnp.exp(jnp.maximum(t, NEG * t) - m)
                        exbuf[e, pl.ds(0, LANES)] = ex
                        return ssum + ex

                    ssum = lax.fori_loop(0, DEG, p2,
                                         jnp.zeros((LANES,), jnp.float32))
                    inv = 1.0 / (ssum + 1e-9)

                    def p3(e, accs):
                        alpha = exbuf[e, pl.ds(0, LANES)] * inv
                        out = list(accs)
                        for g in range(16):
                            t16 = fgat[b, e, g // 8,
                                       pl.ds((g % 8) * LANES, LANES)]
                            va, vb = _bf2f32(t16)
                            out[2 * g] = out[2 * g] + alpha * va
                            out[2 * g + 1] = out[2 * g + 1] + alpha * vb
                        return tuple(out)

                    accs = lax.fori_loop(
                        0, DEG, p3,
                        tuple(jnp.zeros((LANES,), jnp.float32)
                              for _ in range(32)))
                    for c in range(32):
                        obuf[j, pl.ds(c * LANES, LANES)] = accs[c]

            pltpu.sync_copy(obuf,
                            out_hbm.at[pl.ds(base + k * INNER, INNER)])

    return sc_edge


CS0_L1 = 416
CS0_L2 = 416

_sc_edge_l1 = _make_sc_edge_l1_bf16(2, CS0_L1)
_sc_edge_l2 = _make_sc_edge(1, D2, D2, 8, CS0_L2)


def kernel(inputs, row_ptr, col_idx, col_ptr, row_idx, permute,
           W1, al1, ar1, W2, al2, ar2):
    xp = jnp.pad(inputs, ((0, NP_ - N), (0, 0)))
    idx = jnp.pad(
        col_idx, (0, (NP_ + CSMAX) * DEG - N * DEG)).reshape(NP_ + CSMAX, DEG)

    t = np.arange(D1)
    g_, r_ = t // 32, t % 32
    p_, l_ = r_ % 2, r_ // 2
    perm_table = (l_ % 8) * HID + (4 * g_ + 2 * p_ + (l_ >= 8))
    jo = np.arange(D1)
    v_, lo_ = jo // LANES, jo % LANES
    perm_out = (lo_ % 8) * HID + (4 * (v_ // 2) + 2 * (v_ % 2) + (lo_ >= 8))

    w1r = W1.reshape(IN_DIM, H1, HID)
    A1 = jnp.einsum("khd,hd->kh", w1r, al1)
    B1 = jnp.einsum("khd,hd->kh", w1r, ar1)
    WF1 = W1[:, perm_table]
    WLE1 = jnp.concatenate([A1, A1, B1, B1], axis=1)

    F1, LE1 = _mm(xp, WF1, WLE1, f_dtype=jnp.bfloat16)
    F1i = jax.lax.bitcast_convert_type(
        F1.reshape(NP_, D1 // 2, 2), jnp.int32)

    def _hilo(x):
        hi = x.astype(jnp.bfloat16)
        lo = (x - hi.astype(jnp.float32)).astype(jnp.bfloat16)
        return jax.lax.bitcast_convert_type(
            jnp.stack([hi, lo], axis=-1), jnp.int32)

    tab1 = jnp.concatenate(
        [F1i, _hilo(LE1[:, :16]), _hilo(LE1[:, 16:]),
         jnp.zeros((NP_, 96), jnp.int32)], axis=1).reshape(NP_, 3, 128)
    H1o = _sc_edge_l1(tab1, idx)

    W2p = W2[perm_out, :]
    A2 = W2p @ al2[0]
    B2 = W2p @ ar2[0]
    W2p = jnp.pad(W2p, ((0, 0), (0, D2 - C)))
    A2t = jnp.tile(A2[:, None], (1, LANES))
    B2t = jnp.tile(B2[:, None], (1, LANES))
    WF2 = jnp.concatenate(
        [W2p, A2t, B2t,
         jnp.zeros((D1, R2 - D2 - 2 * LANES), jnp.float32)], axis=1)
    WLE2 = jnp.concatenate([A2t, B2t], axis=1)

    F2, LE2 = _mm(H1o, WF2, WLE2)
    H2 = _sc_edge_l2(F2, LE2, idx)
    return H2[:N, :C]
